# Initial kernel scaffold; baseline (speedup 1.0000x reference)
import jax
import jax.numpy as jnp
from jax.experimental import pallas as pl


def kernel(x, s):
    raise NotImplementedError("write your pallas kernel here")


# speedup vs baseline: 1.0639x; 1.0639x over previous
import functools

import jax
import jax.numpy as jnp
import numpy as np
from jax import lax
from jax.experimental import pallas as pl
from jax.experimental.pallas import tpu as pltpu
from jax.experimental.pallas import tpu_sc as plsc

_NC = 2
_NS = 16
_NW = _NC * _NS
_L = 16


def _snap(v, al):
    d = jnp.minimum(jnp.maximum(v / al, -1.0), 1.0)
    db = lax.bitcast_convert_type(d, jnp.int32)
    sb = jnp.bitwise_and(db, np.int32(-0x80000000))
    ab = jnp.bitwise_and(db, np.int32(0x7FFFFFFF))
    isneg = lax.shift_right_logical(db, 31)
    q = jnp.bitwise_and(ab + (np.int32(0x003FFFFF) + isneg), np.int32(0x7F800000))
    q = jnp.maximum(q, np.int32(0x38000000))
    keep = lax.shift_right_arithmetic(np.int32(0x37800000) - isneg - ab, 31)
    q = jnp.bitwise_and(q, keep)
    return lax.bitcast_convert_type(jnp.bitwise_or(q, sb), jnp.float32) * al


@functools.cache
def _build(n_pad):
    per_w = n_pad // _NW
    iters = per_w // _L
    mesh = plsc.VectorSubcoreMesh(core_axis_name="c", subcore_axis_name="s")

    @functools.partial(
        pl.kernel,
        out_type=jax.ShapeDtypeStruct((n_pad,), jnp.float32),
        mesh=mesh,
        scratch_types=[
            pltpu.VMEM((per_w,), jnp.float32),
            pltpu.VMEM((_L,), jnp.float32),
        ],
    )
    def run(x_hbm, a_hbm, o_hbm, buf, avec):
        wid = lax.axis_index("s") * _NC + lax.axis_index("c")
        base = wid * per_w
        pltpu.sync_copy(a_hbm, avec)
        pltpu.sync_copy(x_hbm.at[pl.ds(base, per_w)], buf)
        al = avec[...]

        def body(i, c):
            v = buf[pl.ds(i * _L, _L)]
            buf[pl.ds(i * _L, _L)] = _snap(v, al)
            return c

        lax.fori_loop(0, iters, body, 0)
        pltpu.sync_copy(buf, o_hbm.at[pl.ds(base, per_w)])

    return run


def kernel(x, s):
    n = x.size
    g = 1.0 / (1.0 * n / x.shape[0]) ** 0.5
    sg = s * g
    alpha = (s - sg) + sg
    chunk = _NW * _L
    n_pad = -(-n // chunk) * chunk
    flat = x.reshape(-1)
    if n_pad != n:
        flat = jnp.concatenate([flat, jnp.zeros((n_pad - n,), x.dtype)])
    avec = jnp.broadcast_to(alpha, (_L,))
    out = _build(n_pad)(flat, avec)
    if n_pad != n:
        out = out[:n]
    return out.reshape(x.shape)


# speedup vs baseline: 33.5688x; 31.5515x over previous
import functools

import jax
import jax.numpy as jnp
import numpy as np
from jax import lax
from jax.experimental import pallas as pl
from jax.experimental.pallas import tpu as pltpu
from jax.experimental.pallas import tpu_sc as plsc

_NC = 2
_NS = 16
_NW = _NC * _NS
_L = 16


def _snap(v, al):
    d = jnp.minimum(jnp.maximum(v / al, -1.0), 1.0)
    db = lax.bitcast_convert_type(d, jnp.int32)
    sb = jnp.bitwise_and(db, np.int32(-0x80000000))
    ab = jnp.bitwise_and(db, np.int32(0x7FFFFFFF))
    isneg = lax.shift_right_logical(db, 31)
    q = jnp.bitwise_and(ab + (np.int32(0x003FFFFF) + isneg), np.int32(0x7F800000))
    q = jnp.maximum(q, np.int32(0x38000000))
    keep = lax.shift_right_arithmetic(np.int32(0x37800000) - isneg - ab, 31)
    q = jnp.bitwise_and(q, keep)
    return lax.bitcast_convert_type(jnp.bitwise_or(q, sb), jnp.float32) * al


@functools.cache
def _build(n_pad):
    per_w = n_pad // _NW
    iters = per_w // _L
    mesh = plsc.VectorSubcoreMesh(core_axis_name="c", subcore_axis_name="s")

    @functools.partial(
        pl.kernel,
        out_type=jax.ShapeDtypeStruct((n_pad,), jnp.float32),
        mesh=mesh,
        scratch_types=[
            pltpu.VMEM((per_w,), jnp.float32),
            pltpu.VMEM((_L,), jnp.float32),
        ],
    )
    def run(x_hbm, a_hbm, o_hbm, buf, avec):
        wid = lax.axis_index("s") * _NC + lax.axis_index("c")
        base = wid * per_w
        pltpu.sync_copy(a_hbm, avec)
        pltpu.sync_copy(x_hbm.at[pl.ds(base, per_w)], buf)
        al = avec[...]

        def body(i, c):
            v = buf[pl.ds(i * _L, _L)]
            buf[pl.ds(i * _L, _L)] = _snap(v, al)
            return c

        lax.fori_loop(0, iters, body, 0)
        pltpu.sync_copy(buf, o_hbm.at[pl.ds(base, per_w)])

    return run


def kernel(x, s):
    n = x.size
    g = 1.0 / (1.0 * n / x.shape[0]) ** 0.5
    sg = s * g
    alpha = (s - sg) + sg
    chunk = _NW * _L
    n_pad = -(-n // chunk) * chunk
    perm = (2, 3, 0, 1)
    inv_perm = (2, 3, 0, 1)
    xt = jnp.transpose(x, perm)
    flat = xt.reshape(-1)
    if n_pad != n:
        flat = jnp.concatenate([flat, jnp.zeros((n_pad - n,), x.dtype)])
    avec = jnp.broadcast_to(alpha, (_L,))
    out = _build(n_pad)(flat, avec)
    if n_pad != n:
        out = out[:n]
    return jnp.transpose(out.reshape(xt.shape), inv_perm)


# speedup vs baseline: 56.8945x; 1.6949x over previous
import functools

import jax
import jax.numpy as jnp
import numpy as np
from jax import lax
from jax.experimental import pallas as pl
from jax.experimental.pallas import tpu as pltpu
from jax.experimental.pallas import tpu_sc as plsc

_NC = 2
_NS = 16
_NW = _NC * _NS
_L = 16


def _snap(v, al):
    d = jnp.minimum(jnp.maximum(v / al, -1.0), 1.0)
    db = lax.bitcast_convert_type(d, jnp.int32)
    sb = jnp.bitwise_and(db, np.int32(-0x80000000))
    ab = jnp.bitwise_and(db, np.int32(0x7FFFFFFF))
    isneg = lax.shift_right_logical(db, 31)
    q = jnp.bitwise_and(ab + (np.int32(0x003FFFFF) + isneg), np.int32(0x7F800000))
    q = jnp.maximum(q, np.int32(0x38000000))
    keep = lax.shift_right_arithmetic(np.int32(0x37800000) - isneg - ab, 31)
    q = jnp.bitwise_and(q, keep)
    return lax.bitcast_convert_type(jnp.bitwise_or(q, sb), jnp.float32) * al


@functools.cache
def _build(n_pad):
    per_w = n_pad // _NW
    iters = per_w // _L
    mesh = plsc.VectorSubcoreMesh(core_axis_name="c", subcore_axis_name="s")

    @functools.partial(
        pl.kernel,
        out_type=jax.ShapeDtypeStruct((n_pad,), jnp.float32),
        mesh=mesh,
        scratch_types=[
            pltpu.VMEM((per_w,), jnp.float32),
            pltpu.VMEM((_L,), jnp.float32),
        ],
    )
    def run(x_hbm, a_hbm, o_hbm, buf, avec):
        wid = lax.axis_index("s") * _NC + lax.axis_index("c")
        base = wid * per_w
        pltpu.sync_copy(a_hbm, avec)
        pltpu.sync_copy(x_hbm.at[pl.ds(base, per_w)], buf)
        al = avec[...]

        @plsc.parallel_loop(0, per_w, step=_L, unroll=8)
        def _loop(i):
            v = buf[pl.ds(i, _L)]
            buf[pl.ds(i, _L)] = _snap(v, al)
        pltpu.sync_copy(buf, o_hbm.at[pl.ds(base, per_w)])

    return run


def kernel(x, s):
    n = x.size
    g = 1.0 / (1.0 * n / x.shape[0]) ** 0.5
    sg = s * g
    alpha = (s - sg) + sg
    chunk = _NW * _L
    n_pad = -(-n // chunk) * chunk
    perm = (2, 3, 0, 1)
    inv_perm = (2, 3, 0, 1)
    xt = jnp.transpose(x, perm)
    flat = xt.reshape(-1)
    if n_pad != n:
        flat = jnp.concatenate([flat, jnp.zeros((n_pad - n,), x.dtype)])
    avec = jnp.broadcast_to(alpha, (_L,))
    out = _build(n_pad)(flat, avec)
    if n_pad != n:
        out = out[:n]
    return jnp.transpose(out.reshape(xt.shape), inv_perm)


# speedup vs baseline: 83.8033x; 1.4730x over previous
import functools

import jax
import jax.numpy as jnp
import numpy as np
from jax import lax
from jax.experimental import pallas as pl
from jax.experimental.pallas import tpu as pltpu
from jax.experimental.pallas import tpu_sc as plsc

_NC = 2
_NS = 16
_NW = _NC * _NS
_L = 16


def _snap(v, al):
    d = jnp.minimum(jnp.maximum(v / al, -1.0), 1.0)
    db = lax.bitcast_convert_type(d, jnp.int32)
    sb = jnp.bitwise_and(db, np.int32(-0x80000000))
    ab = jnp.bitwise_and(db, np.int32(0x7FFFFFFF))
    isneg = lax.shift_right_logical(db, 31)
    q = jnp.bitwise_and(ab + (np.int32(0x003FFFFF) + isneg), np.int32(0x7F800000))
    q = jnp.maximum(q, np.int32(0x38000000))
    keep = lax.shift_right_arithmetic(np.int32(0x37800000) - isneg - ab, 31)
    q = jnp.bitwise_and(q, keep)
    return lax.bitcast_convert_type(jnp.bitwise_or(q, sb), jnp.float32) * al


@functools.cache
def _build_rows(n_rows, n_cols):
    rows_w = n_rows // _NW
    cols_v = n_cols // _L
    mesh = plsc.VectorSubcoreMesh(core_axis_name="c", subcore_axis_name="s")

    @functools.partial(
        pl.kernel,
        out_type=jax.ShapeDtypeStruct((n_rows, n_cols), jnp.float32),
        mesh=mesh,
        scratch_types=[
            pltpu.VMEM((rows_w, n_cols), jnp.float32),
            pltpu.VMEM((_L,), jnp.float32),
        ],
    )
    def run(x_hbm, a_hbm, o_hbm, buf, avec):
        wid = lax.axis_index("s") * _NC + lax.axis_index("c")
        base = wid * rows_w
        pltpu.sync_copy(a_hbm, avec)
        pltpu.sync_copy(x_hbm.at[pl.ds(base, rows_w)], buf)
        al = avec[...]

        @plsc.parallel_loop(0, rows_w, step=1, unroll=2)
        def _loop(r):
            for c in range(cols_v):
                v = buf[r, pl.ds(c * _L, _L)]
                buf[r, pl.ds(c * _L, _L)] = _snap(v, al)

        pltpu.sync_copy(buf, o_hbm.at[pl.ds(base, rows_w)])

    return run


@functools.cache
def _build_flat(n_pad):
    per_w = n_pad // _NW
    mesh = plsc.VectorSubcoreMesh(core_axis_name="c", subcore_axis_name="s")

    @functools.partial(
        pl.kernel,
        out_type=jax.ShapeDtypeStruct((n_pad,), jnp.float32),
        mesh=mesh,
        scratch_types=[
            pltpu.VMEM((per_w,), jnp.float32),
            pltpu.VMEM((_L,), jnp.float32),
        ],
    )
    def run(x_hbm, a_hbm, o_hbm, buf, avec):
        wid = lax.axis_index("s") * _NC + lax.axis_index("c")
        base = wid * per_w
        pltpu.sync_copy(a_hbm, avec)
        pltpu.sync_copy(x_hbm.at[pl.ds(base, per_w)], buf)
        al = avec[...]

        @plsc.parallel_loop(0, per_w, step=_L, unroll=8)
        def _loop(i):
            v = buf[pl.ds(i, _L)]
            buf[pl.ds(i, _L)] = _snap(v, al)

        pltpu.sync_copy(buf, o_hbm.at[pl.ds(base, per_w)])

    return run


def kernel(x, s):
    n = x.size
    g = 1.0 / (1.0 * n / x.shape[0]) ** 0.5
    sg = s * g
    alpha = (s - sg) + sg
    avec = jnp.broadcast_to(alpha, (_L,))

    perm = (2, 3, 0, 1)
    inv_perm = (2, 3, 0, 1)
    xt = jnp.transpose(x, perm)
    a, b, r, c = xt.shape
    if c % _L == 0 and (a * b * r) % _NW == 0:
        x2 = xt.reshape(a * b * r, c)
        out = _build_rows(a * b * r, c)(x2, avec)
        return jnp.transpose(out.reshape(xt.shape), inv_perm)

    chunk = _NW * _L
    n_pad = -(-n // chunk) * chunk
    flat = xt.reshape(-1)
    if n_pad != n:
        flat = jnp.concatenate([flat, jnp.zeros((n_pad - n,), x.dtype)])
    out = _build_flat(n_pad)(flat, avec)
    if n_pad != n:
        out = out[:n]
    return jnp.transpose(out.reshape(xt.shape), inv_perm)


# speedup vs baseline: 89.1475x; 1.0638x over previous
import functools

import jax
import jax.numpy as jnp
import numpy as np
from jax import lax
from jax.experimental import pallas as pl
from jax.experimental.pallas import tpu as pltpu
from jax.experimental.pallas import tpu_sc as plsc

_NC = 2
_NS = 16
_NW = _NC * _NS
_L = 16


def _snap(v, al):
    d = jnp.minimum(jnp.maximum(v / al, -1.0), 1.0)
    db = lax.bitcast_convert_type(d, jnp.int32)
    sb = jnp.bitwise_and(db, np.int32(-0x80000000))
    ab = jnp.bitwise_and(db, np.int32(0x7FFFFFFF))
    isneg = lax.shift_right_logical(db, 31)
    q = jnp.bitwise_and(ab + (np.int32(0x003FFFFF) + isneg), np.int32(0x7F800000))
    q = jnp.maximum(q, np.int32(0x38000000))
    keep = lax.shift_right_arithmetic(np.int32(0x37800000) - isneg - ab, 31)
    q = jnp.bitwise_and(q, keep)
    return lax.bitcast_convert_type(jnp.bitwise_or(q, sb), jnp.float32) * al


def _snap_mul(v, ral, al):
    db = lax.bitcast_convert_type(v * ral, jnp.int32)
    sb = jnp.bitwise_and(db, np.int32(-0x80000000))
    ab = jnp.bitwise_and(db, np.int32(0x7FFFFFFF))
    t = ab + lax.shift_right_logical(db, 31)
    q = jnp.bitwise_and(t + np.int32(0x003FFFFF), np.int32(0x7F800000))
    q = jnp.maximum(q, np.int32(0x38000000))
    q = jnp.minimum(q, np.int32(0x3F800000))
    keep = lax.shift_right_arithmetic(np.int32(0x37800000) - t, 31)
    q = jnp.bitwise_and(q, keep)
    return lax.bitcast_convert_type(jnp.bitwise_or(q, sb), jnp.float32) * al


@functools.cache
def _build_rows(n_rows, n_cols, n_chunks, g):
    rows_w = n_rows // _NW
    rows_c = rows_w // n_chunks
    cols_v = n_cols // _L
    mesh = plsc.VectorSubcoreMesh(core_axis_name="c", subcore_axis_name="s")

    @functools.partial(
        pl.kernel,
        out_type=jax.ShapeDtypeStruct((n_rows, n_cols), jnp.float32),
        mesh=mesh,
        scratch_types=[
            pltpu.VMEM((2, rows_c, n_cols), jnp.float32),
            pltpu.VMEM((2, rows_c, n_cols), jnp.float32),
            pltpu.VMEM((_L,), jnp.float32),
            pltpu.SemaphoreType.DMA,
            pltpu.SemaphoreType.DMA,
            pltpu.SemaphoreType.DMA,
            pltpu.SemaphoreType.DMA,
        ],
    )
    def run(x_hbm, s_hbm, o_hbm, ibuf, obuf, svec, si0, si1, so0, so1):
        sin = (si0, si1)
        sout = (so0, so1)
        wid = lax.axis_index("s") * _NC + lax.axis_index("c")
        base = wid * rows_w

        def in_cp(k):
            b = k & 1
            return pltpu.make_async_copy(
                x_hbm.at[pl.ds(base + k * rows_c, rows_c)], ibuf.at[b], sin[b])

        def out_cp(k):
            b = k & 1
            return pltpu.make_async_copy(
                obuf.at[b], o_hbm.at[pl.ds(base + k * rows_c, rows_c)], sout[b])

        in_cp(0).start()
        if n_chunks > 1:
            in_cp(1).start()
        pltpu.sync_copy(s_hbm, svec)
        sv = svec[...]
        sg = sv * g
        al = (sv - sg) + sg
        ral = 1.0 / al

        for k in range(n_chunks):
            b = k & 1
            in_cp(k).wait()
            if k >= 2:
                out_cp(k - 2).wait()

            @plsc.parallel_loop(0, rows_c, step=1, unroll=2)
            def _loop(r, _b=b):
                for c in range(cols_v):
                    v = ibuf[_b, r, pl.ds(c * _L, _L)]
                    obuf[_b, r, pl.ds(c * _L, _L)] = _snap_mul(v, ral, al)

            out_cp(k).start()
            if k + 2 < n_chunks:
                in_cp(k + 2).start()

        if n_chunks >= 2:
            out_cp(n_chunks - 2).wait()
        out_cp(n_chunks - 1).wait()

    return run


@functools.cache
def _build_flat(n_pad):
    per_w = n_pad // _NW
    mesh = plsc.VectorSubcoreMesh(core_axis_name="c", subcore_axis_name="s")

    @functools.partial(
        pl.kernel,
        out_type=jax.ShapeDtypeStruct((n_pad,), jnp.float32),
        mesh=mesh,
        scratch_types=[
            pltpu.VMEM((per_w,), jnp.float32),
            pltpu.VMEM((_L,), jnp.float32),
        ],
    )
    def run(x_hbm, a_hbm, o_hbm, buf, avec):
        wid = lax.axis_index("s") * _NC + lax.axis_index("c")
        base = wid * per_w
        pltpu.sync_copy(a_hbm, avec)
        pltpu.sync_copy(x_hbm.at[pl.ds(base, per_w)], buf)
        al = avec[...]

        @plsc.parallel_loop(0, per_w, step=_L, unroll=8)
        def _loop(i):
            v = buf[pl.ds(i, _L)]
            buf[pl.ds(i, _L)] = _snap(v, al)

        pltpu.sync_copy(buf, o_hbm.at[pl.ds(base, per_w)])

    return run


def kernel(x, s):
    n = x.size
    g = 1.0 / (1.0 * n / x.shape[0]) ** 0.5

    perm = (2, 3, 0, 1)
    inv_perm = (2, 3, 0, 1)
    xt = jnp.transpose(x, perm)
    a, b, r, c = xt.shape
    n_rows = a * b * r
    if c % _L == 0 and n_rows % (_NW * 8) == 0:
        rows_w = n_rows // _NW
        n_chunks = 1
        for nc in (3, 4, 2):
            if rows_w % (8 * nc) == 0:
                n_chunks = nc
                break
        s16 = jnp.broadcast_to(s, (_L,))
        out = _build_rows(n_rows, c, n_chunks, g)(xt.reshape(n_rows, c), s16)
        return jnp.transpose(out.reshape(xt.shape), inv_perm)

    sg = s * g
    alpha = (s - sg) + sg
    avec = jnp.broadcast_to(alpha, (_L,))
    chunk = _NW * _L
    n_pad = -(-n // chunk) * chunk
    flat = xt.reshape(-1)
    if n_pad != n:
        flat = jnp.concatenate([flat, jnp.zeros((n_pad - n,), x.dtype)])
    out = _build_flat(n_pad)(flat, avec)
    if n_pad != n:
        out = out[:n]
    return jnp.transpose(out.reshape(xt.shape), inv_perm)
